# baseline (speedup 1.0000x reference)
import functools

import jax
import jax.numpy as jnp
from jax import lax
from jax.experimental import pallas as pl
from jax.experimental.pallas import tpu as pltpu
from jax.experimental.pallas import tpu_sc as plsc

N = 10000
NP = 10240
E = 320000
D = 128

NC = 2
NS = 16
NW = NC * NS
EPW = E // NW
CH = 16
NCHUNK = EPW // CH
RPT = NP // NS
NRING = 4


def _sc_segment_sum(neigh_vecs, cmb, wts):
    @functools.partial(
        pl.kernel,
        out_type=(
            jax.ShapeDtypeStruct((NP, D), jnp.float32),
            jax.ShapeDtypeStruct((NP, D), jnp.float32),
        ),
        mesh=plsc.VectorSubcoreMesh(core_axis_name="c", subcore_axis_name="s"),
        scratch_types=[
            pltpu.VMEM_SHARED((NP, D), jnp.float32),
            pltpu.VMEM((CH, D), jnp.float32),
            pltpu.VMEM((CH, D), jnp.float32),
            pltpu.VMEM((CH, D), jnp.float32),
            pltpu.VMEM((CH, D), jnp.float32),
            [pltpu.VMEM((2, CH), jnp.int32)] * NRING,
            [pltpu.VMEM((1, CH), jnp.float32)] * NRING,
            [pltpu.SemaphoreType.DMA] * 2,
            [pltpu.SemaphoreType.DMA] * 2,
            [pltpu.SemaphoreType.DMA] * NRING,
            [pltpu.SemaphoreType.DMA] * NRING,
        ],
    )
    def body(neigh, cmb_h, wts_h, p0, p1, acc, grows0, grows1, srows0,
             srows1, cbuf, wbuf, gsem, ssem, csem, wsem):
        cid = lax.axis_index("c")
        sid = lax.axis_index("s")
        wid = sid * NC + cid

        grows = (grows0, grows1)
        srows = (srows0, srows1)

        def idx_start(k, s):
            pltpu.async_copy(cmb_h.at[wid, k], cbuf[s], csem[s])
            pltpu.async_copy(wts_h.at[wid, k], wbuf[s], wsem[s])

        def idx_wait(k, s):
            pltpu.make_async_copy(cmb_h.at[wid, k], cbuf[s], csem[s]).wait()
            pltpu.make_async_copy(wts_h.at[wid, k], wbuf[s], wsem[s]).wait()

        def gather_start(s, b):
            pltpu.async_copy(neigh.at[cbuf[s].at[0]], grows[b], gsem[b])

        def gather_wait(s, b):
            pltpu.make_async_copy(neigh.at[cbuf[s].at[0]], grows[b],
                                  gsem[b]).wait()

        def scatter_start(s, b):
            pltpu.async_copy(srows[b], acc.at[cbuf[s].at[1]], ssem[b],
                             add=True)

        def scatter_wait(s, b):
            pltpu.make_async_copy(srows[b], acc.at[cbuf[s].at[1]],
                                  ssem[b]).wait()

        def scale(s, b):
            @pl.loop(0, CH // 16)
            def _scale(g):
                wv = wbuf[s][0, pl.ds(g * 16, 16)]
                for l in range(16):
                    w = wv[l]
                    e = g * 16 + l
                    for j in range(D // 16):
                        sl = pl.ds(j * 16, 16)
                        srows[b][e, sl] = grows[b][e, sl] * w

        def step(k, s, b, first=False):
            gather_wait(s, b)
            if not first:
                scatter_wait((s + 2) % NRING, b)
            idx_start(k + 2, (s + 2) % NRING)
            scale(s, b)
            scatter_start(s, b)
            idx_wait(k + 2, (s + 2) % NRING)
            gather_start((s + 2) % NRING, b)

        idx_start(0, 0)
        idx_start(1, 1)

        @pl.loop(0, CH)
        def _zero(r):
            for j in range(D // 16):
                srows0[r, pl.ds(j * 16, 16)] = jnp.zeros((16,), jnp.float32)

        for kk in range(RPT // CH):
            pltpu.sync_copy(srows0, acc.at[pl.ds(sid * RPT + kk * CH, CH)])

        idx_wait(0, 0)
        gather_start(0, 0)
        idx_wait(1, 1)
        gather_start(1, 1)
        plsc.subcore_barrier()

        step(0, 0, 0, first=True)
        step(1, 1, 1, first=True)

        @pl.loop(2, NCHUNK - 3, step=NRING)
        def _main(k4):
            for i in range(NRING):
                step(k4 + i, (2 + i) % NRING, i % 2)

        kt = NCHUNK - 3
        gather_wait(2, 0)
        scatter_wait(0, 0)
        idx_start(kt + 2, 0)
        scale(2, 0)
        scatter_start(2, 0)
        idx_wait(kt + 2, 0)
        gather_start(0, 0)

        gather_wait(3, 1)
        scatter_wait(1, 1)
        scale(3, 1)
        scatter_start(3, 1)

        gather_wait(0, 0)
        scatter_wait(2, 0)
        scale(0, 0)
        scatter_start(0, 0)

        scatter_wait(3, 1)
        scatter_wait(0, 0)
        plsc.subcore_barrier()

        r0 = sid * RPT

        @pl.when(cid == 0)
        def _():
            pltpu.sync_copy(acc.at[pl.ds(r0, RPT)], p0.at[pl.ds(r0, RPT)])

        @pl.when(cid == 1)
        def _():
            pltpu.sync_copy(acc.at[pl.ds(r0, RPT)], p1.at[pl.ds(r0, RPT)])

    return body(neigh_vecs, cmb, wts)


BN = 1000


def _tc_body(self_ref, p0_ref, p1_ref, sw_ref, nw_ref, out_ref):
    fs = jnp.dot(self_ref[...], sw_ref[...], preferred_element_type=jnp.float32)
    nm = p0_ref[...] + p1_ref[...]
    fn = jnp.dot(nm, nw_ref[...], preferred_element_type=jnp.float32)
    out_ref[:, :D] = jnp.maximum(fs, 0.0)
    out_ref[:, D:] = jnp.maximum(fn, 0.0)


def _tc_dense(self_vecs, p0, p1, self_weights, neigh_weights):
    return pl.pallas_call(
        _tc_body,
        grid=(N // BN,),
        in_specs=[
            pl.BlockSpec((BN, D), lambda i: (i, 0)),
            pl.BlockSpec((BN, D), lambda i: (i, 0)),
            pl.BlockSpec((BN, D), lambda i: (i, 0)),
            pl.BlockSpec((D, D), lambda i: (0, 0)),
            pl.BlockSpec((D, D), lambda i: (0, 0)),
        ],
        out_specs=pl.BlockSpec((BN, 2 * D), lambda i: (i, 0)),
        out_shape=jax.ShapeDtypeStruct((N, 2 * D), jnp.float32),
    )(self_vecs, p0, p1, self_weights, neigh_weights)


def kernel(neigh_vecs, self_vecs, edge_index, edge_weight, neigh_weights,
           self_weights):
    src = edge_index[0].astype(jnp.int32).reshape(NW, NCHUNK, CH)
    dst = edge_index[1].astype(jnp.int32).reshape(NW, NCHUNK, CH)
    cmb = jnp.stack([src, dst], axis=2)
    wts = edge_weight.reshape(NW, NCHUNK, 1, CH)
    p0, p1 = _sc_segment_sum(neigh_vecs, cmb, wts)
    return _tc_dense(self_vecs, p0, p1, self_weights, neigh_weights)


# speedup vs baseline: 1.8640x; 1.8640x over previous
import functools

import jax
import jax.numpy as jnp
from jax import lax
from jax.experimental import pallas as pl
from jax.experimental.pallas import tpu as pltpu
from jax.experimental.pallas import tpu_sc as plsc

N = 10000
NP = 10240
E = 320000
D = 128

NC = 2
NS = 16
NW = NC * NS
EPW = E // NW
CH = 80
NCHUNK = EPW // CH
RPT = NP // NS
NRING = 4


def _sc_segment_sum(neigh_vecs, cmb, wts):
    @functools.partial(
        pl.kernel,
        out_type=(
            jax.ShapeDtypeStruct((NP, D), jnp.float32),
            jax.ShapeDtypeStruct((NP, D), jnp.float32),
        ),
        mesh=plsc.VectorSubcoreMesh(core_axis_name="c", subcore_axis_name="s"),
        scratch_types=[
            pltpu.VMEM_SHARED((NP, D), jnp.float32),
            pltpu.VMEM((CH, D), jnp.float32),
            pltpu.VMEM((CH, D), jnp.float32),
            pltpu.VMEM((CH, D), jnp.float32),
            [pltpu.VMEM((2, CH), jnp.int32)] * NRING,
            [pltpu.VMEM((1, CH), jnp.float32)] * NRING,
            [pltpu.SemaphoreType.DMA] * 2,
            pltpu.SemaphoreType.DMA,
            [pltpu.SemaphoreType.DMA] * NRING,
            [pltpu.SemaphoreType.DMA] * NRING,
        ],
    )
    def body(neigh, cmb_h, wts_h, p0, p1, acc, grows0, grows1, srows,
             cbuf, wbuf, gsem, ssem, csem, wsem):
        cid = lax.axis_index("c")
        sid = lax.axis_index("s")
        wid = sid * NC + cid

        grows = (grows0, grows1)

        def idx_start(k, s):
            pltpu.async_copy(cmb_h.at[wid, k], cbuf[s], csem[s])
            pltpu.async_copy(wts_h.at[wid, k], wbuf[s], wsem[s])

        def idx_wait(k, s):
            pltpu.make_async_copy(cmb_h.at[wid, k], cbuf[s], csem[s]).wait()
            pltpu.make_async_copy(wts_h.at[wid, k], wbuf[s], wsem[s]).wait()

        def gather_start(s, b):
            pltpu.async_copy(neigh.at[cbuf[s].at[0]], grows[b], gsem[b])

        def gather_wait(s, b):
            pltpu.make_async_copy(neigh.at[cbuf[s].at[0]], grows[b],
                                  gsem[b]).wait()

        def scatter_start(s):
            pltpu.async_copy(srows, acc.at[cbuf[s].at[1]], ssem, add=True)

        def scatter_wait(s):
            pltpu.make_async_copy(srows, acc.at[cbuf[s].at[1]], ssem).wait()

        def scale(s, b):
            @pl.loop(0, CH // 16)
            def _scale(g):
                wv = wbuf[s][0, pl.ds(g * 16, 16)]
                for l in range(16):
                    w = wv[l]
                    e = g * 16 + l
                    for j in range(D // 16):
                        sl = pl.ds(j * 16, 16)
                        srows[e, sl] = grows[b][e, sl] * w

        def step(k, s, b, first=False, starts=True):
            gather_wait(s, b)
            if starts:
                idx_start(k + 2, (s + 2) % NRING)
            if not first:
                scatter_wait((s + 3) % NRING)
            scale(s, b)
            scatter_start(s)
            if starts:
                idx_wait(k + 2, (s + 2) % NRING)
                gather_start((s + 2) % NRING, b)

        idx_start(0, 0)
        idx_start(1, 1)

        @pl.loop(0, CH)
        def _zero(r):
            for j in range(D // 16):
                srows[r, pl.ds(j * 16, 16)] = jnp.zeros((16,), jnp.float32)

        for kk in range(RPT // CH):
            pltpu.sync_copy(srows, acc.at[pl.ds(sid * RPT + kk * CH, CH)])

        idx_wait(0, 0)
        gather_start(0, 0)
        idx_wait(1, 1)
        gather_start(1, 1)
        plsc.subcore_barrier()

        step(0, 0, 0, first=True)

        @pl.loop(1, NCHUNK - 4, step=NRING)
        def _main(k4):
            for i in range(NRING):
                step(k4 + i, (1 + i) % NRING, (1 + i) % 2)

        step(NCHUNK - 4, 1, 1)
        step(NCHUNK - 3, 2, 0)
        step(NCHUNK - 2, 3, 1, starts=False)
        step(NCHUNK - 1, 0, 0, starts=False)
        scatter_wait(0)
        plsc.subcore_barrier()

        r0 = sid * RPT

        @pl.when(cid == 0)
        def _():
            pltpu.sync_copy(acc.at[pl.ds(r0, RPT)], p0.at[pl.ds(r0, RPT)])

        @pl.when(cid == 1)
        def _():
            pltpu.sync_copy(acc.at[pl.ds(r0, RPT)], p1.at[pl.ds(r0, RPT)])

    return body(neigh_vecs, cmb, wts)


BN = 1000


def _tc_body(self_ref, p0_ref, p1_ref, sw_ref, nw_ref, out_ref):
    fs = jnp.dot(self_ref[...], sw_ref[...], preferred_element_type=jnp.float32)
    nm = p0_ref[...] + p1_ref[...]
    fn = jnp.dot(nm, nw_ref[...], preferred_element_type=jnp.float32)
    out_ref[:, :D] = jnp.maximum(fs, 0.0)
    out_ref[:, D:] = jnp.maximum(fn, 0.0)


def _tc_dense(self_vecs, p0, p1, self_weights, neigh_weights):
    return pl.pallas_call(
        _tc_body,
        grid=(N // BN,),
        in_specs=[
            pl.BlockSpec((BN, D), lambda i: (i, 0)),
            pl.BlockSpec((BN, D), lambda i: (i, 0)),
            pl.BlockSpec((BN, D), lambda i: (i, 0)),
            pl.BlockSpec((D, D), lambda i: (0, 0)),
            pl.BlockSpec((D, D), lambda i: (0, 0)),
        ],
        out_specs=pl.BlockSpec((BN, 2 * D), lambda i: (i, 0)),
        out_shape=jax.ShapeDtypeStruct((N, 2 * D), jnp.float32),
    )(self_vecs, p0, p1, self_weights, neigh_weights)


def kernel(neigh_vecs, self_vecs, edge_index, edge_weight, neigh_weights,
           self_weights):
    src = edge_index[0].astype(jnp.int32).reshape(NW, NCHUNK, CH)
    dst = edge_index[1].astype(jnp.int32).reshape(NW, NCHUNK, CH)
    cmb = jnp.stack([src, dst], axis=2)
    wts = edge_weight.reshape(NW, NCHUNK, 1, CH)
    p0, p1 = _sc_segment_sum(neigh_vecs, cmb, wts)
    return _tc_dense(self_vecs, p0, p1, self_weights, neigh_weights)
